# baseline (speedup 1.0000x reference)
import functools

import jax
import jax.numpy as jnp
from jax import lax
from jax.experimental import pallas as pl
from jax.experimental.pallas import tpu as pltpu
from jax.experimental.pallas import tpu_sc as plsc

N = 4096
NW = 32
RPW = N // NW
CH = 8
NCH = RPW // CH
L = 16

_mesh = plsc.VectorSubcoreMesh(core_axis_name="c", subcore_axis_name="s")
_cparams = pltpu.CompilerParams(needs_layout_passes=False)


@functools.partial(
    pl.kernel,
    out_type=jax.ShapeDtypeStruct((N,), jnp.float32),
    mesh=_mesh,
    compiler_params=_cparams,
    scratch_types=[
        pltpu.VMEM((CH, N), jnp.float32),
        pltpu.VMEM((CH, N), jnp.float32),
        pltpu.VMEM((RPW,), jnp.float32),
        pltpu.SemaphoreType.DMA,
        pltpu.SemaphoreType.DMA,
    ],
)
def _rowsum_k(mat_hbm, rs_hbm, buf0, buf1, rs_v, sem0, sem1):
    wid = lax.axis_index("s") * 2 + lax.axis_index("c")
    base = wid * RPW
    bufs = (buf0, buf1)
    sems = (sem0, sem1)
    zero = jnp.zeros((L,), jnp.float32)
    lanes = lax.iota(jnp.int32, L)

    cps = [None, None]
    cps[0] = pltpu.async_copy(mat_hbm.at[pl.ds(base, CH)], buf0, sem0)
    for k in range(NCH):
        if k + 1 < NCH:
            nb = (k + 1) % 2
            cps[nb] = pltpu.async_copy(
                mat_hbm.at[pl.ds(base + (k + 1) * CH, CH)], bufs[nb], sems[nb]
            )
        cps[k % 2].wait()
        buf = bufs[k % 2]
        g16 = (k // 2) * L
        lane0 = (k % 2) * CH

        def row_body(r, _, buf=buf, g16=g16, lane0=lane0):
            def col_body(j, accs):
                a0, a1, a2, a3 = accs
                c0 = pl.multiple_of(j * 256, 256)
                for q in range(4):
                    o = c0 + q * 64
                    a0 = a0 + buf[r, pl.ds(o, L)]
                    a1 = a1 + buf[r, pl.ds(o + 16, L)]
                    a2 = a2 + buf[r, pl.ds(o + 32, L)]
                    a3 = a3 + buf[r, pl.ds(o + 48, L)]
                return (a0, a1, a2, a3)

            a0, a1, a2, a3 = lax.fori_loop(
                0, 16, col_body, (zero, zero, zero, zero)
            )
            s = jnp.sum((a0 + a1) + (a2 + a3))
            old = rs_v[pl.ds(g16, L)]
            rs_v[pl.ds(g16, L)] = jnp.where(lanes == lane0 + r, s, old)
            return 0

        lax.fori_loop(0, CH, row_body, 0)

    pltpu.sync_copy(rs_v, rs_hbm.at[pl.ds(base, RPW)])


@functools.partial(
    pl.kernel,
    out_type=jax.ShapeDtypeStruct((N - 1,), jnp.float32),
    mesh=_mesh,
    compiler_params=_cparams,
    scratch_types=[
        pltpu.VMEM((N,), jnp.float32),
        pltpu.VMEM((L,), jnp.int32),
        pltpu.VMEM((N,), jnp.float32),
    ],
)
def _compact_k(last_hbm, rs_hbm, out_hbm, rs_v, last_v, st_v):
    wid = lax.axis_index("s") * 2 + lax.axis_index("c")

    @pl.when(wid == 0)
    def _():
        pltpu.sync_copy(rs_hbm, rs_v)
        pltpu.sync_copy(last_hbm, last_v)
        last_vec = last_v[...]
        lanes = lax.iota(jnp.int32, L)
        neg1 = jnp.full((L,), -1.0, jnp.float32)

        def fill_body(i, _):
            st_v[pl.ds(pl.multiple_of(i * L, L), L)] = neg1
            return 0

        lax.fori_loop(0, N // L, fill_body, 0)

        def cbody(i, off):
            v = rs_v[pl.ds(pl.multiple_of(i * L, L), L)]
            v = jnp.where(lanes + i * L == last_vec, -1.0, v)
            m = v != -1.0
            mi = jnp.where(m, jnp.int32(1), jnp.int32(0))
            c = plsc.cumsum(mi)
            pos = (off + c) - 1
            plsc.store_scatter(st_v, [pos], v, mask=m)
            return off + jnp.sum(mi)

        lax.fori_loop(0, N // L, cbody, jnp.int32(0))
        pltpu.sync_copy(st_v.at[pl.ds(0, N - 1)], out_hbm)


def kernel(first_cam_trap, last_cam_trap, cond_mat):
    last16 = jnp.broadcast_to(last_cam_trap.astype(jnp.int32), (L,))
    rs = _rowsum_k(cond_mat)
    out = _compact_k(last16, rs)
    return out.reshape(-1, 1)
